# baseline (speedup 1.0000x reference)
import jax
import jax.numpy as jnp
from jax.experimental import pallas as pl

_BLOCK_ROWS = 5000


def _linear_body(x_ref, w_ref, o_ref):
    o_ref[...] = jax.lax.dot_general(
        x_ref[...],
        w_ref[...],
        dimension_numbers=(((1,), (1,)), ((), ())),
        preferred_element_type=jnp.float32,
    )


def kernel(features, W_fc):
    n, k = features.shape
    h = W_fc.shape[0]
    grid = n // _BLOCK_ROWS
    return pl.pallas_call(
        _linear_body,
        grid=(grid,),
        in_specs=[
            pl.BlockSpec((_BLOCK_ROWS, k), lambda i: (i, 0)),
            pl.BlockSpec((h, k), lambda i: (0, 0)),
        ],
        out_specs=pl.BlockSpec((_BLOCK_ROWS, h), lambda i: (i, 0)),
        out_shape=jax.ShapeDtypeStruct((n, h), jnp.float32),
    )(features, W_fc)


# speedup vs baseline: 1.0672x; 1.0672x over previous
import jax
import jax.numpy as jnp
from jax.experimental import pallas as pl

_BLOCK_ROWS = 20000


def _linear_body(x_ref, w_ref, o_ref):
    o_ref[...] = jax.lax.dot_general(
        x_ref[...],
        w_ref[...],
        dimension_numbers=(((1,), (1,)), ((), ())),
        preferred_element_type=jnp.float32,
    )


def kernel(features, W_fc):
    n, k = features.shape
    h = W_fc.shape[0]
    grid = n // _BLOCK_ROWS
    return pl.pallas_call(
        _linear_body,
        grid=(grid,),
        in_specs=[
            pl.BlockSpec((_BLOCK_ROWS, k), lambda i: (i, 0)),
            pl.BlockSpec((h, k), lambda i: (0, 0)),
        ],
        out_specs=pl.BlockSpec((_BLOCK_ROWS, h), lambda i: (i, 0)),
        out_shape=jax.ShapeDtypeStruct((n, h), jnp.float32),
    )(features, W_fc)


# speedup vs baseline: 6.3024x; 5.9054x over previous
import jax
import jax.numpy as jnp
from jax.experimental import pallas as pl
from jax.experimental.pallas import tpu as pltpu

_BLOCK_N = 8192


def _linear_t_body(w_ref, x_ref, o_ref):
    o_ref[...] = jax.lax.dot_general(
        w_ref[...],
        x_ref[...],
        dimension_numbers=(((0,), (0,)), ((), ())),
        preferred_element_type=jnp.float32,
    )


def kernel(features, W_fc):
    n, k = features.shape
    h = W_fc.shape[0]
    ft = features.T
    wt = W_fc.T
    grid = pl.cdiv(n, _BLOCK_N)
    out_t = pl.pallas_call(
        _linear_t_body,
        grid=(grid,),
        in_specs=[
            pl.BlockSpec((k, h), lambda i: (0, 0)),
            pl.BlockSpec((k, _BLOCK_N), lambda i: (0, i)),
        ],
        out_specs=pl.BlockSpec((h, _BLOCK_N), lambda i: (0, i)),
        out_shape=jax.ShapeDtypeStruct((h, n), jnp.float32),
        compiler_params=pltpu.CompilerParams(
            dimension_semantics=("arbitrary",),
        ),
    )(wt, ft)
    return out_t.T


# speedup vs baseline: 7.8027x; 1.2381x over previous
import jax
import jax.numpy as jnp
from jax.experimental import pallas as pl
from jax.experimental.pallas import tpu as pltpu

_BLOCK_N = 16384


def _linear_t_body(w_ref, x_ref, o_ref):
    o_ref[...] = jax.lax.dot_general(
        w_ref[...],
        x_ref[...],
        dimension_numbers=(((0,), (0,)), ((), ())),
        preferred_element_type=jnp.float32,
    )


def kernel(features, W_fc):
    n, k = features.shape
    h = W_fc.shape[0]
    ft = features.T
    wt = W_fc.T
    grid = pl.cdiv(n, _BLOCK_N)
    out_t = pl.pallas_call(
        _linear_t_body,
        grid=(grid,),
        in_specs=[
            pl.BlockSpec((k, h), lambda i: (0, 0)),
            pl.BlockSpec((k, _BLOCK_N), lambda i: (0, i)),
        ],
        out_specs=pl.BlockSpec((h, _BLOCK_N), lambda i: (0, i)),
        out_shape=jax.ShapeDtypeStruct((h, n), jnp.float32),
        compiler_params=pltpu.CompilerParams(
            dimension_semantics=("arbitrary",),
        ),
    )(wt, ft)
    return out_t.T


# speedup vs baseline: 8.5016x; 1.0896x over previous
import jax
import jax.numpy as jnp
from jax.experimental import pallas as pl
from jax.experimental.pallas import tpu as pltpu

_BLOCK_N = 32768


def _linear_t_body(w_ref, x_ref, o_ref):
    o_ref[...] = jax.lax.dot_general(
        w_ref[...],
        x_ref[...],
        dimension_numbers=(((0,), (0,)), ((), ())),
        preferred_element_type=jnp.float32,
    )


def kernel(features, W_fc):
    n, k = features.shape
    h = W_fc.shape[0]
    ft = features.T
    wt = W_fc.T
    grid = pl.cdiv(n, _BLOCK_N)
    out_t = pl.pallas_call(
        _linear_t_body,
        grid=(grid,),
        in_specs=[
            pl.BlockSpec((k, h), lambda i: (0, 0)),
            pl.BlockSpec((k, _BLOCK_N), lambda i: (0, i)),
        ],
        out_specs=pl.BlockSpec((h, _BLOCK_N), lambda i: (0, i)),
        out_shape=jax.ShapeDtypeStruct((h, n), jnp.float32),
        compiler_params=pltpu.CompilerParams(
            dimension_semantics=("arbitrary",),
        ),
    )(wt, ft)
    return out_t.T
